# baseline (speedup 1.0000x reference)
import functools

import jax
import jax.numpy as jnp
from jax import lax
from jax.experimental import pallas as pl
from jax.experimental.pallas import tpu as pltpu
from jax.experimental.pallas import tpu_sc as plsc

B = 16384
D = 64

_NC = 2
_NS = 16
_NW = _NC * _NS
_BPW = B // _NW
_PH = 128
_NGRP = _PH // 16


def _sc_gather_body(idx_hbm, tab_hbm, out_hbm, idx_v, grp_v, sem):
    wid = lax.axis_index("s") * _NC + lax.axis_index("c")
    base = wid * _BPW
    pltpu.sync_copy(idx_hbm.at[pl.ds(base, _BPW)], idx_v)

    for p in range(_BPW // _PH):
        def group(g, carry):
            chunk = idx_v[pl.ds(p * _PH + g * 16, 16)]
            for j in range(16):
                s = chunk[j]
                g8 = pl.multiple_of((s // 8) * 8, 8)
                jj = g * 16 + j
                pltpu.async_copy(tab_hbm.at[pl.ds(g8, 8)],
                                 grp_v.at[pl.ds(jj * 8, 8)], sem)
            return carry

        lax.fori_loop(0, _NGRP, group, 0)
        pltpu.make_async_copy(tab_hbm.at[pl.ds(0, _PH * 8)], grp_v,
                              sem).wait()
        pltpu.sync_copy(grp_v, out_hbm.at[pl.ds((base + p * _PH) * 8,
                                                _PH * 8)])


@functools.lru_cache(maxsize=1)
def _sc_gather():
    return pl.kernel(
        _sc_gather_body,
        out_type=jax.ShapeDtypeStruct((B * 8, D), jnp.bfloat16),
        mesh=plsc.VectorSubcoreMesh(core_axis_name="c", subcore_axis_name="s"),
        scratch_types=[
            pltpu.VMEM((_BPW,), jnp.int32),
            pltpu.VMEM((_PH * 8, D), jnp.bfloat16),
            pltpu.SemaphoreType.DMA,
        ],
    )


_BLK = 512


def _mlp_body(ug_ref, vg_ref, mu_ref, mv_ref, w1u_ref, w1v_ref, b_ref,
              w2_ref, o_ref):
    u = jnp.zeros((_BLK, D), jnp.float32)
    v = jnp.zeros((_BLK, D), jnp.float32)
    for k in range(8):
        u = u + ug_ref[:, k, :].astype(jnp.float32) * mu_ref[:, k:k + 1]
        v = v + vg_ref[:, k, :].astype(jnp.float32) * mv_ref[:, k:k + 1]
    h = (jnp.dot(u, w1u_ref[...], preferred_element_type=jnp.float32)
         + jnp.dot(v, w1v_ref[...], preferred_element_type=jnp.float32)
         + b_ref[...])
    h = jnp.maximum(h, 0.0)
    o_ref[...] = jnp.dot(h, w2_ref[...],
                         preferred_element_type=jnp.float32,
                         precision=lax.Precision.HIGHEST)


_mlp = pl.pallas_call(
    _mlp_body,
    grid=(B // _BLK,),
    in_specs=[
        pl.BlockSpec((_BLK, 8, D), lambda i: (i, 0, 0)),
        pl.BlockSpec((_BLK, 8, D), lambda i: (i, 0, 0)),
        pl.BlockSpec((_BLK, 8), lambda i: (i, 0)),
        pl.BlockSpec((_BLK, 8), lambda i: (i, 0)),
        pl.BlockSpec((D, D), lambda i: (0, 0)),
        pl.BlockSpec((D, D), lambda i: (0, 0)),
        pl.BlockSpec((1, D), lambda i: (0, 0)),
        pl.BlockSpec((D, 1), lambda i: (0, 0)),
    ],
    out_specs=pl.BlockSpec((_BLK, 1), lambda i: (i, 0)),
    out_shape=jax.ShapeDtypeStruct((B, 1), jnp.float32),
)


def kernel(x, W_table, H_table, lin1_w, lin1_b, lin2_w):
    uidx = x[:, 0]
    iidx = x[:, 1]
    vg = _sc_gather()(iidx, H_table.astype(jnp.bfloat16)).reshape(B, 8, D)
    ug = _sc_gather()(uidx, W_table.astype(jnp.bfloat16)).reshape(B, 8, D)
    mu = jax.nn.one_hot(uidx % 8, 8, dtype=jnp.float32)
    mv = jax.nn.one_hot(iidx % 8, 8, dtype=jnp.float32)
    w1t = lin1_w.T
    return _mlp(ug, vg, mu, mv, w1t[:D], w1t[D:], lin1_b.reshape(1, D),
                lin2_w.T)


# speedup vs baseline: 1.3056x; 1.3056x over previous
import functools

import jax
import jax.numpy as jnp
from jax import lax
from jax.experimental import pallas as pl
from jax.experimental.pallas import tpu as pltpu
from jax.experimental.pallas import tpu_sc as plsc

B = 16384
D = 64

_NC = 2
_NS = 16
_NW = _NC * _NS
_BPW = B // _NW
_NGRP = _BPW // 16


def _sc_gather_body(idx_hbm, tab_hbm, out_hbm, idx_v, rows_v, sem):
    wid = lax.axis_index("s") * _NC + lax.axis_index("c")
    base = wid * _BPW
    pltpu.sync_copy(idx_hbm.at[pl.ds(base, _BPW)], idx_v)

    def group(g, carry):
        chunk = idx_v[pl.ds(g * 16, 16)]
        for j in range(16):
            s = chunk[j]
            pltpu.async_copy(tab_hbm.at[pl.ds(s, 1)],
                             rows_v.at[pl.ds(g * 16 + j, 1)], sem)
        return carry

    lax.fori_loop(0, _NGRP, group, 0)
    pltpu.make_async_copy(tab_hbm.at[pl.ds(0, _BPW)], rows_v, sem).wait()
    pltpu.sync_copy(rows_v, out_hbm.at[pl.ds(base, _BPW)])


@functools.lru_cache(maxsize=1)
def _sc_gather():
    return pl.kernel(
        _sc_gather_body,
        out_type=jax.ShapeDtypeStruct((B, D), jnp.float32),
        mesh=plsc.VectorSubcoreMesh(core_axis_name="c", subcore_axis_name="s"),
        scratch_types=[
            pltpu.VMEM((_BPW,), jnp.int32),
            pltpu.VMEM((_BPW, D), jnp.float32),
            pltpu.SemaphoreType.DMA,
        ],
    )


_BLK = 2048


def _mlp_body(u_ref, v_ref, w1u_ref, w1v_ref, b_ref, w2_ref, o_ref):
    h = (jnp.dot(u_ref[...], w1u_ref[...],
                 preferred_element_type=jnp.float32,
                 precision=lax.Precision.HIGHEST)
         + jnp.dot(v_ref[...], w1v_ref[...],
                   preferred_element_type=jnp.float32,
                   precision=lax.Precision.HIGHEST)
         + b_ref[...])
    h = jnp.maximum(h, 0.0)
    o_ref[...] = jnp.dot(h, w2_ref[...],
                         preferred_element_type=jnp.float32,
                         precision=lax.Precision.HIGHEST)


_mlp = pl.pallas_call(
    _mlp_body,
    grid=(B // _BLK,),
    in_specs=[
        pl.BlockSpec((_BLK, D), lambda i: (i, 0)),
        pl.BlockSpec((_BLK, D), lambda i: (i, 0)),
        pl.BlockSpec((D, D), lambda i: (0, 0)),
        pl.BlockSpec((D, D), lambda i: (0, 0)),
        pl.BlockSpec((1, D), lambda i: (0, 0)),
        pl.BlockSpec((D, 1), lambda i: (0, 0)),
    ],
    out_specs=pl.BlockSpec((_BLK, 1), lambda i: (i, 0)),
    out_shape=jax.ShapeDtypeStruct((B, 1), jnp.float32),
)


def kernel(x, W_table, H_table, lin1_w, lin1_b, lin2_w):
    uidx = x[:, 0]
    iidx = x[:, 1]
    v_emb = _sc_gather()(iidx, H_table)
    u_emb = _sc_gather()(uidx, W_table)
    w1t = lin1_w.T
    return _mlp(u_emb, v_emb, w1t[:D], w1t[D:], lin1_b.reshape(1, D),
                lin2_w.T)
